# baseline (speedup 1.0000x reference)
import functools

import jax
import jax.numpy as jnp
from jax import lax
from jax.experimental import pallas as pl
from jax.experimental.pallas import tpu as pltpu
from jax.experimental.pallas import tpu_sc as plsc

_NC = 2
_NS = 16
_NW = _NC * _NS
_CS = 8


def _sc_body(x_hbm, emb_hbm, out_hbm, xbuf, ebuf):
    S, B, D = x_hbm.shape
    rows_per_w = S // _NW
    n_chunks = rows_per_w // _CS
    nd = D // 16

    c = lax.axis_index("c")
    s = lax.axis_index("s")
    wid = s * _NC + c
    row0 = wid * rows_per_w

    def chunk(k, _):
        r = row0 + k * _CS
        pltpu.sync_copy(x_hbm.at[pl.ds(r, _CS)], xbuf)
        pltpu.sync_copy(emb_hbm.at[pl.ds(r, _CS)], ebuf)

        def body(t, _):
            si = t // nd
            j = (t % nd) * 16
            e = ebuf[si, pl.ds(j, 16)]
            for b in range(B):
                xbuf[si, b, pl.ds(j, 16)] += e
            return 0

        lax.fori_loop(0, _CS * nd, body, 0)
        pltpu.sync_copy(xbuf, out_hbm.at[pl.ds(r, _CS)])
        return 0

    lax.fori_loop(0, n_chunks, chunk, 0)


def kernel(x, emb_table):
    S, B, D = x.shape
    mesh = plsc.VectorSubcoreMesh(core_axis_name="c", subcore_axis_name="s")
    f = functools.partial(
        pl.kernel,
        out_type=jax.ShapeDtypeStruct((S, B, D), x.dtype),
        mesh=mesh,
        scratch_types=[
            pltpu.VMEM((_CS, B, D), jnp.float32),
            pltpu.VMEM((_CS, D), jnp.float32),
        ],
    )(_sc_body)
    return f(x, emb_table)


# speedup vs baseline: 2.0199x; 2.0199x over previous
import functools

import jax
import jax.numpy as jnp
from jax import lax
from jax.experimental import pallas as pl
from jax.experimental.pallas import tpu as pltpu
from jax.experimental.pallas import tpu_sc as plsc

_NC = 2
_NS = 16
_NW = _NC * _NS
_CS = 8
_SC_ROWS = 512


def _add_kernel(x_ref, e_ref, o_ref):
    o_ref[...] = x_ref[...] + e_ref[...][:, None, :]


def _sc_body(x_hbm, emb_hbm, out_hbm, xbuf, ebuf):
    S2, B, D = out_hbm.shape
    row_base = x_hbm.shape[0] - S2
    rows_per_w = S2 // _NW
    n_chunks = rows_per_w // _CS
    nd = D // 16

    c = lax.axis_index("c")
    s = lax.axis_index("s")
    wid = s * _NC + c
    row0 = wid * rows_per_w

    def chunk(k, _):
        r = row0 + k * _CS
        pltpu.sync_copy(x_hbm.at[pl.ds(row_base + r, _CS)], xbuf)
        pltpu.sync_copy(emb_hbm.at[pl.ds(row_base + r, _CS)], ebuf)

        def body(t, _):
            si = t // nd
            j = (t % nd) * 16
            e = ebuf[si, pl.ds(j, 16)]
            for b in range(B):
                xbuf[si, b, pl.ds(j, 16)] += e
            return 0

        lax.fori_loop(0, _CS * nd, body, 0)
        pltpu.sync_copy(xbuf, out_hbm.at[pl.ds(r, _CS)])
        return 0

    lax.fori_loop(0, n_chunks, chunk, 0)


def kernel(x, emb_table):
    S, B, D = x.shape
    S1 = S - _SC_ROWS
    BS = 512

    tc_out = pl.pallas_call(
        _add_kernel,
        grid=(S1 // BS,),
        in_specs=[
            pl.BlockSpec((BS, B, D), lambda i: (i, 0, 0)),
            pl.BlockSpec((BS, D), lambda i: (i, 0)),
        ],
        out_specs=pl.BlockSpec((BS, B, D), lambda i: (i, 0, 0)),
        out_shape=jax.ShapeDtypeStruct((S, B, D), x.dtype),
    )(x, emb_table)

    sc_part = functools.partial(
        pl.kernel,
        out_type=jax.ShapeDtypeStruct((_SC_ROWS, B, D), x.dtype),
        mesh=plsc.VectorSubcoreMesh(core_axis_name="c", subcore_axis_name="s"),
        scratch_types=[
            pltpu.VMEM((_CS, B, D), jnp.float32),
            pltpu.VMEM((_CS, D), jnp.float32),
        ],
    )(_sc_body)(x, emb_table)

    return lax.dynamic_update_slice(tc_out, sc_part, (S1, 0, 0))
